# baseline (speedup 1.0000x reference)
import functools

import jax
import jax.numpy as jnp
from jax import lax
from jax.experimental import pallas as pl
from jax.experimental.pallas import tpu as pltpu
from jax.experimental.pallas import tpu_sc as plsc

V = 1000000
D = 32
B = 16384

NC = 2
NS = 16
L = 16
NW = NC * NS
BPW = B // NW
CHUNK = 128
NCHUNK = BPW // CHUNK
NBLK = BPW // L

_mesh = plsc.VectorSubcoreMesh(core_axis_name="c", subcore_axis_name="s")


@functools.partial(
    pl.kernel,
    mesh=_mesh,
    out_type=(
        jax.ShapeDtypeStruct((B,), jnp.float32),
        jax.ShapeDtypeStruct((B,), jnp.float32),
    ),
    scratch_types=[
        pltpu.VMEM((BPW,), jnp.int32),
        pltpu.VMEM((BPW,), jnp.int32),
        pltpu.VMEM((BPW,), jnp.int32),
        pltpu.VMEM((BPW, D), jnp.float32),
        pltpu.VMEM((BPW, D), jnp.float32),
        pltpu.VMEM((BPW, D), jnp.float32),
        pltpu.VMEM((BPW,), jnp.float32),
        pltpu.VMEM((BPW,), jnp.float32),
        pltpu.SemaphoreType.DMA,
    ],
    compiler_params=pltpu.CompilerParams(
        needs_layout_passes=False, use_tc_tiling_on_sc=False
    ),
)
def _sc_dots(iword, owords, nwords, emb_i, emb_o, od_hbm, nd_hbm,
             iidx, oidx, nidx, ivec, ovec, nvec, od_v, nd_v, sem):
    wid = lax.axis_index("s") * NC + lax.axis_index("c")
    base = wid * BPW

    pltpu.sync_copy(iword.at[pl.ds(base, BPW)], iidx)
    pltpu.sync_copy(owords.at[pl.ds(base, BPW)], oidx)
    pltpu.sync_copy(nwords.at[pl.ds(base, BPW)], nidx)

    copies = []
    for j in range(NCHUNK):
        sl = pl.ds(j * CHUNK, CHUNK)
        copies.append(pltpu.async_copy(emb_i.at[iidx.at[sl]], ivec.at[sl], sem))
        copies.append(pltpu.async_copy(emb_o.at[oidx.at[sl]], ovec.at[sl], sem))
        copies.append(pltpu.async_copy(emb_o.at[nidx.at[sl]], nvec.at[sl], sem))
    for c in copies:
        c.wait()

    lanes = lax.broadcasted_iota(jnp.int32, (L,), 0)

    def blk_body(b, carry):
        acc_o = jnp.zeros((L,), jnp.float32)
        acc_n = jnp.zeros((L,), jnp.float32)
        for k in range(L):
            r = b * L + k
            iv0 = ivec[r, pl.ds(0, L)]
            iv1 = ivec[r, pl.ds(L, L)]
            ov0 = ovec[r, pl.ds(0, L)]
            ov1 = ovec[r, pl.ds(L, L)]
            nv0 = nvec[r, pl.ds(0, L)]
            nv1 = nvec[r, pl.ds(L, L)]
            so = jnp.sum(iv0 * ov0 + iv1 * ov1)
            sn = jnp.sum(iv0 * nv0 + iv1 * nv1)
            acc_o = jnp.where(lanes == k, so, acc_o)
            acc_n = jnp.where(lanes == k, sn, acc_n)
        od_v[pl.ds(b * L, L)] = acc_o
        nd_v[pl.ds(b * L, L)] = acc_n
        return carry

    lax.fori_loop(0, NBLK, blk_body, 0)

    pltpu.sync_copy(od_v, od_hbm.at[pl.ds(base, BPW)])
    pltpu.sync_copy(nd_v, nd_hbm.at[pl.ds(base, BPW)])


def _loss_body(od_ref, nd_ref, out_ref):
    od = od_ref[...]
    nd = nd_ref[...]
    lso = jnp.minimum(od, 0.0) - jnp.log1p(jnp.exp(-jnp.abs(od)))
    x = -nd
    lsn = jnp.minimum(x, 0.0) - jnp.log1p(jnp.exp(-jnp.abs(x)))
    out_ref[0, 0] = -(jnp.sum(lso) + jnp.sum(lsn)) / B


_tc_loss = pl.pallas_call(
    _loss_body,
    out_shape=jax.ShapeDtypeStruct((1, 1), jnp.float32),
    out_specs=pl.BlockSpec(memory_space=pltpu.SMEM),
)


def kernel(iword, owords, nwords, emb_i, emb_o):
    iword = iword.astype(jnp.int32)
    owords = owords.astype(jnp.int32)
    nwords = nwords.astype(jnp.int32)
    od, nd = _sc_dots(iword, owords, nwords, emb_i, emb_o)
    out = _tc_loss(od.reshape(128, 128), nd.reshape(128, 128))
    return out[0, 0]
